# baseline (speedup 1.0000x reference)
import functools

import jax
import jax.numpy as jnp
from jax import lax
from jax.experimental import pallas as pl
from jax.experimental.pallas import tpu as pltpu
from jax.experimental.pallas import tpu_sc as plsc

N = 50000
E = 1600000
F_IN = 128
H = 32
HH = H // 2

NC = 2
NS = 16
NW = NC * NS
CH = 128
E_ROWS = 12544
E_PAD = E_ROWS * CH
ROWS_PER_S = E_ROWS // NS
KB = 8
OUTER_S = ROWS_PER_S // KB
ROWS_PER_W = E_ROWS // NW
N_TILE = 3128
N_PAD = NS * N_TILE


@functools.lru_cache(maxsize=None)
def _mesh():
    return plsc.VectorSubcoreMesh(
        core_axis_name="c", subcore_axis_name="s", num_cores=NC, num_subcores=NS
    )


_RB = 1000


def _tc1_body(x_ref, wi_ref, bi_ref, wt_ref, wb_ref, b0_ref,
              p_ref, qlo_ref, qhi_ref):
    h0 = jnp.maximum(
        jnp.dot(x_ref[...], wi_ref[...], preferred_element_type=jnp.float32)
        + bi_ref[...], 0.0)
    p_ref[...] = jnp.dot(h0, wt_ref[...],
                         preferred_element_type=jnp.float32) + b0_ref[...]
    q = jnp.dot(h0, wb_ref[...], preferred_element_type=jnp.float32)
    qlo_ref[...] = q[:, :HH]
    qhi_ref[...] = q[:, HH:]


def _tc_mid_body(p_ref, sa_ref, sb_ref, da_ref, db_ref, wt_ref, wb_ref, b_ref,
                 p2_ref, qlo_ref, qhi_ref):
    deg = jnp.maximum(da_ref[...] + db_ref[...], 1.0)[:, 0:1]
    s = jnp.concatenate([sa_ref[...], sb_ref[...]], axis=1)
    h = jnp.maximum(p_ref[...] + s / deg, 0.0)
    p2_ref[...] = jnp.dot(h, wt_ref[...],
                          preferred_element_type=jnp.float32) + b_ref[...]
    q = jnp.dot(h, wb_ref[...], preferred_element_type=jnp.float32)
    qlo_ref[...] = q[:, :HH]
    qhi_ref[...] = q[:, HH:]


def _tc_fin_body(p_ref, sa_ref, sb_ref, da_ref, db_ref, wt_ref, wb_ref, b_ref,
                 a_ref, b2_ref):
    deg = jnp.maximum(da_ref[...] + db_ref[...], 1.0)[:, 0:1]
    s = jnp.concatenate([sa_ref[...], sb_ref[...]], axis=1)
    h = jnp.maximum(p_ref[...] + s / deg, 0.0)
    a_ref[...] = jnp.dot(h, wt_ref[...],
                         preferred_element_type=jnp.float32) + b_ref[...]
    b2_ref[...] = jnp.dot(h, wb_ref[...], preferred_element_type=jnp.float32)


def _node_spec(w):
    return pl.BlockSpec((_RB, w), lambda i: (i, 0))


def _full_spec(r, w):
    return pl.BlockSpec((r, w), lambda i: (0, 0))


def _tc1(x, wi, bi, wt, wb, b0):
    return pl.pallas_call(
        _tc1_body,
        grid=(N // _RB,),
        in_specs=[
            _node_spec(F_IN), _full_spec(F_IN, H), _full_spec(1, H),
            _full_spec(H, H), _full_spec(H, H), _full_spec(1, H),
        ],
        out_specs=[_node_spec(H), _node_spec(HH), _node_spec(HH)],
        out_shape=[
            jax.ShapeDtypeStruct((N, H), jnp.float32),
            jax.ShapeDtypeStruct((N, HH), jnp.float32),
            jax.ShapeDtypeStruct((N, HH), jnp.float32),
        ],
    )(x, wi, bi, wt, wb, b0)


def _tc_mid(p, sa, sb, da, db, wt, wb, b):
    return pl.pallas_call(
        _tc_mid_body,
        grid=(N // _RB,),
        in_specs=[
            _node_spec(H), _node_spec(HH), _node_spec(HH), _node_spec(8),
            _node_spec(8),
            _full_spec(H, H), _full_spec(H, H), _full_spec(1, H),
        ],
        out_specs=[_node_spec(H), _node_spec(HH), _node_spec(HH)],
        out_shape=[
            jax.ShapeDtypeStruct((N, H), jnp.float32),
            jax.ShapeDtypeStruct((N, HH), jnp.float32),
            jax.ShapeDtypeStruct((N, HH), jnp.float32),
        ],
    )(p, sa, sb, da, db, wt, wb, b)


def _tc_fin(p, sa, sb, da, db, wt, wb, b):
    return pl.pallas_call(
        _tc_fin_body,
        grid=(N // _RB,),
        in_specs=[
            _node_spec(H), _node_spec(HH), _node_spec(HH), _node_spec(8),
            _node_spec(8),
            _full_spec(H, H), _full_spec(H, H), _full_spec(1, H),
        ],
        out_specs=[_node_spec(H), _node_spec(H)],
        out_shape=[
            jax.ShapeDtypeStruct((N, H), jnp.float32),
            jax.ShapeDtypeStruct((N, H), jnp.float32),
        ],
    )(p, sa, sb, da, db, wt, wb, b)


DEPTH = 4
STEPS_S = 104
STEPS_F = 394


def _seg_body(row2d, col2d, q_lo, q_hi, z16, *refs):
    n = DEPTH
    s_out = refs[0]
    refs = refs[1:]
    ridx = refs[0:n]
    cidx = refs[n:2 * n]
    rows = refs[2 * n:3 * n]
    acc = refs[3 * n]
    refs = refs[3 * n + 1:]
    semi = refs[0:n]
    semg = refs[n:2 * n]
    sems = refs[2 * n:3 * n]

    c = lax.axis_index("c")
    s = lax.axis_index("s")
    lo = s * N_TILE
    pltpu.sync_copy(z16.at[pl.ds(lo, N_TILE)], acc.at[pl.ds(lo, N_TILE)])
    plsc.subcore_barrier()
    base = s * ROWS_PER_S

    def fire_idx(b, m):
        r0 = base + b * KB
        pltpu.async_copy(row2d.at[pl.ds(r0, KB)], ridx[m], semi[m])
        pltpu.async_copy(col2d.at[pl.ds(r0, KB)], cidx[m], semi[m])

    def wait_idx(m):
        pltpu.make_async_copy(row2d.at[pl.ds(base, KB)], ridx[m], semi[m]).wait()
        pltpu.make_async_copy(col2d.at[pl.ds(base, KB)], cidx[m], semi[m]).wait()

    def fire_gathers(m):
        @pl.when(c == 0)
        def _():
            for j in range(KB):
                pltpu.async_copy(q_lo.at[cidx[m].at[j]], rows[m].at[j], semg[m])

        @pl.when(c == 1)
        def _():
            for j in range(KB):
                pltpu.async_copy(q_hi.at[cidx[m].at[j]], rows[m].at[j], semg[m])

    def wait_gathers(m):
        for j in range(KB):
            pltpu.make_async_copy(
                q_lo.at[cidx[m].at[j]], rows[m].at[j], semg[m]).wait()

    def fire_scatters(m):
        for j in range(KB):
            pltpu.async_copy(rows[m].at[j], acc.at[ridx[m].at[j]], sems[m],
                             add=True)

    def wait_scatters(m):
        for j in range(KB):
            pltpu.make_async_copy(
                rows[m].at[j], acc.at[ridx[m].at[j]], sems[m]).wait()

    fire_idx(0, 0)

    def loop_body(k, carry):
        i0 = k * DEPTH
        for u in range(DEPTH):
            i = i0 + u
            m = u
            m1 = (u + 1) % DEPTH
            mp = (u - 1) % DEPTH

            @pl.when(jnp.logical_and(i >= 3, i <= OUTER_S + 2))
            def _():
                wait_scatters(m1)

            @pl.when(i < OUTER_S)
            def _():
                wait_idx(m)
                fire_gathers(m)

            @pl.when(i + 1 < OUTER_S)
            def _():
                fire_idx(i + 1, m1)

            @pl.when(jnp.logical_and(i >= 1, i <= OUTER_S))
            def _():
                wait_gathers(mp)
                fire_scatters(mp)
        return carry

    lax.fori_loop(0, STEPS_S // DEPTH, loop_body, 0)
    plsc.subcore_barrier()
    pltpu.sync_copy(acc.at[pl.ds(lo, N_TILE)], s_out.at[c, pl.ds(lo, N_TILE)])


def _seg_scratch():
    sc = []
    sc += [pltpu.VMEM((KB, CH), jnp.int32) for _ in range(DEPTH)]
    sc += [pltpu.VMEM((KB, CH), jnp.int32) for _ in range(DEPTH)]
    sc += [pltpu.VMEM((KB, CH, HH), jnp.float32) for _ in range(DEPTH)]
    sc += [pltpu.VMEM_SHARED((N_PAD, HH), jnp.float32)]
    sc += [pltpu.SemaphoreType.DMA for _ in range(3 * DEPTH)]
    return sc


@functools.lru_cache(maxsize=None)
def _segsum():
    return pl.kernel(
        _seg_body,
        out_type=[jax.ShapeDtypeStruct((NC, N_PAD, HH), jnp.float32)],
        mesh=_mesh(),
        compiler_params=pltpu.CompilerParams(use_tc_tiling_on_sc=False, needs_layout_passes=False),
        scratch_types=_seg_scratch(),
    )


OUTER_D = ROWS_PER_W // KB
STEPS_D = 52


def _deg_body(row2d, z4, ones4, d_out, *refs):
    n = DEPTH
    ridx = refs[0:n]
    ones_v = refs[n]
    dacc = refs[n + 1]
    semi = refs[n + 2:2 * n + 2]
    sems = refs[2 * n + 2:3 * n + 2]

    c = lax.axis_index("c")
    s = lax.axis_index("s")
    wid = s * NC + c
    lo = s * N_TILE
    pltpu.sync_copy(z4.at[pl.ds(lo, N_TILE)], dacc.at[pl.ds(lo, N_TILE)])
    pltpu.sync_copy(ones4, ones_v)
    plsc.subcore_barrier()
    base = wid * ROWS_PER_W

    def fire_idx(b, m):
        r0 = base + b * KB
        pltpu.async_copy(row2d.at[pl.ds(r0, KB)], ridx[m], semi[m])

    def wait_idx(m):
        pltpu.make_async_copy(row2d.at[pl.ds(base, KB)], ridx[m], semi[m]).wait()

    def fire_scatters(m):
        for j in range(KB):
            pltpu.async_copy(ones_v, dacc.at[ridx[m].at[j]], sems[m], add=True)

    def wait_scatters(m):
        for j in range(KB):
            pltpu.make_async_copy(ones_v, dacc.at[ridx[m].at[j]], sems[m]).wait()

    fire_idx(0, 0)

    def loop_body(k, carry):
        i0 = k * DEPTH
        for u in range(DEPTH):
            i = i0 + u
            m = u
            m1 = (u + 1) % DEPTH

            @pl.when(jnp.logical_and(i >= 3, i <= OUTER_D + 2))
            def _():
                wait_scatters(m1)

            @pl.when(i < OUTER_D)
            def _():
                wait_idx(m)
                fire_scatters(m)

            @pl.when(i + 1 < OUTER_D)
            def _():
                fire_idx(i + 1, m1)
        return carry

    lax.fori_loop(0, STEPS_D // DEPTH, loop_body, 0)
    plsc.subcore_barrier()
    pltpu.sync_copy(dacc.at[pl.ds(lo, N_TILE)], d_out.at[c, pl.ds(lo, N_TILE)])


@functools.lru_cache(maxsize=None)
def _deg():
    return pl.kernel(
        _deg_body,
        out_type=jax.ShapeDtypeStruct((NC, N_PAD, 8), jnp.float32),
        mesh=_mesh(),
        compiler_params=pltpu.CompilerParams(use_tc_tiling_on_sc=False, needs_layout_passes=False),
        scratch_types=(
            [pltpu.VMEM((KB, CH), jnp.int32) for _ in range(DEPTH)]
            + [pltpu.VMEM((CH, 8), jnp.float32)]
            + [pltpu.VMEM_SHARED((N_PAD, 8), jnp.float32)]
            + [pltpu.SemaphoreType.DMA for _ in range(2 * DEPTH)]
        ),
    )


KF = 2
NB_F = ROWS_PER_W // KF
STEPS_FB = 198


def _flux_body(row2d, col2d, a_t, b_t, w2b, b2b, out,
               ridx0, ridx1, cidx0, cidx1, ra0, ra1, rb0, rb1, fb0, fb1,
               w2v, b2v, semi0, semi1, semg0, semg1, semw0, semw1):
    ridx = (ridx0, ridx1)
    cidx = (cidx0, cidx1)
    ra = (ra0, ra1)
    rb = (rb0, rb1)
    fbuf = (fb0, fb1)
    semi = (semi0, semi1)
    semg = (semg0, semg1)
    semw = (semw0, semw1)
    c = lax.axis_index("c")
    s = lax.axis_index("s")
    wid = s * NC + c
    pltpu.sync_copy(w2b, w2v)
    pltpu.sync_copy(b2b, b2v)
    base = wid * ROWS_PER_W
    iotas = [lax.iota(jnp.int32, 16) + g * 16 for g in range(CH // 16)]
    w2rows = [w2v[jj] for jj in range(H)]
    b2 = b2v[...]

    def fire_idx(kb, p):
        r0 = base + kb * KF
        pltpu.async_copy(row2d.at[pl.ds(r0, KF)], ridx[p], semi[p])
        pltpu.async_copy(col2d.at[pl.ds(r0, KF)], cidx[p], semi[p])

    def wait_idx(p):
        pltpu.make_async_copy(row2d.at[pl.ds(base, KF)], ridx[p], semi[p]).wait()
        pltpu.make_async_copy(col2d.at[pl.ds(base, KF)], cidx[p], semi[p]).wait()

    def fire_gathers(p):
        for r in range(KF):
            pltpu.async_copy(a_t.at[ridx[p].at[r]], ra[p].at[r], semg[p])
            pltpu.async_copy(b_t.at[cidx[p].at[r]], rb[p].at[r], semg[p])

    def wait_gathers(p):
        for r in range(KF):
            pltpu.make_async_copy(a_t.at[ridx[p].at[r]], ra[p].at[r],
                                  semg[p]).wait()
            pltpu.make_async_copy(b_t.at[cidx[p].at[r]], rb[p].at[r],
                                  semg[p]).wait()

    def compute_batch(p, kb):
        for r in range(KF):
            rar = ra[p].at[r]
            rbr = rb[p].at[r]
            for g in range(CH // 16):
                gi = iotas[g]
                acc = b2
                for jj in range(H):
                    jv = jnp.full((16,), jj, jnp.int32)
                    va = plsc.load_gather(rar, [gi, jv])
                    vb = plsc.load_gather(rbr, [gi, jv])
                    sv = jnp.maximum(va + vb, 0.0)
                    acc = acc + sv * w2rows[jj]
                fbuf[p][r, pl.ds(g * 16, 16)] = acc
        pltpu.async_copy(fbuf[p], out.at[pl.ds(base + kb * KF, KF)], semw[p])

    def wait_write(p):
        pltpu.make_async_copy(fbuf[p], out.at[pl.ds(base, KF)], semw[p]).wait()

    fire_idx(0, 0)

    def loop_body(k0, carry):
        for u in range(2):
            k = k0 * 2 + u
            p = u
            pp = 1 - u

            @pl.when(k < NB_F)
            def _():
                wait_idx(p)
                fire_gathers(p)

            @pl.when(jnp.logical_and(k >= 1, k <= NB_F))
            def _():
                wait_gathers(pp)

                @pl.when(k >= 3)
                def _():
                    wait_write(pp)
                compute_batch(pp, k - 1)

            @pl.when(k + 1 < NB_F)
            def _():
                fire_idx(k + 1, pp)
        return carry

    lax.fori_loop(0, STEPS_FB // 2, loop_body, 0)
    wait_write(0)
    wait_write(1)


@functools.lru_cache(maxsize=None)
def _flux():
    return pl.kernel(
        _flux_body,
        out_type=jax.ShapeDtypeStruct((E_ROWS, CH), jnp.float32),
        mesh=_mesh(),
        compiler_params=pltpu.CompilerParams(use_tc_tiling_on_sc=False, needs_layout_passes=False),
        scratch_types=[
            pltpu.VMEM((KF, CH), jnp.int32),
            pltpu.VMEM((KF, CH), jnp.int32),
            pltpu.VMEM((KF, CH), jnp.int32),
            pltpu.VMEM((KF, CH), jnp.int32),
            pltpu.VMEM((KF, CH, H), jnp.float32),
            pltpu.VMEM((KF, CH, H), jnp.float32),
            pltpu.VMEM((KF, CH, H), jnp.float32),
            pltpu.VMEM((KF, CH, H), jnp.float32),
            pltpu.VMEM((KF, CH), jnp.float32),
            pltpu.VMEM((KF, CH), jnp.float32),
            pltpu.VMEM((H, 16), jnp.float32),
            pltpu.VMEM((16,), jnp.float32),
            pltpu.SemaphoreType.DMA,
            pltpu.SemaphoreType.DMA,
            pltpu.SemaphoreType.DMA,
            pltpu.SemaphoreType.DMA,
            pltpu.SemaphoreType.DMA,
            pltpu.SemaphoreType.DMA,
        ],
    )


def kernel(node_features, edge_index, W_in, b_in, W_up0, b_up0,
           W_up1, b_up1, W_e1, b_e1, W_e2, b_e2):
    row = edge_index[0].astype(jnp.int32)
    col = edge_index[1].astype(jnp.int32)
    pad = E_PAD - E
    row2d = jnp.concatenate(
        [row, jnp.full((pad,), N, jnp.int32)]).reshape(E_ROWS, CH)
    col2d = jnp.concatenate(
        [col, jnp.zeros((pad,), jnp.int32)]).reshape(E_ROWS, CH)

    z16 = jnp.zeros((N_PAD, HH), jnp.float32)
    z4 = jnp.zeros((N_PAD, 8), jnp.float32)
    ones4 = jnp.ones((CH, 8), jnp.float32)

    bi = b_in.reshape(1, H)
    b0 = b_up0.reshape(1, H)
    b1 = b_up1.reshape(1, H)
    be1 = b_e1.reshape(1, H)

    p1, q1lo, q1hi = _tc1(node_features, W_in, bi, W_up0[:H], W_up0[H:], b0)
    dpart = _deg()(row2d, z4, ones4)
    (s1,) = _segsum()(row2d, col2d, q1lo, q1hi, z16)
    da, db = dpart[0, :N], dpart[1, :N]
    p2, q2lo, q2hi = _tc_mid(p1, s1[0, :N], s1[1, :N], da, db,
                             W_up1[:H], W_up1[H:], b1)
    (s2,) = _segsum()(row2d, col2d, q2lo, q2hi, z16)
    a_t, b_t = _tc_fin(p2, s2[0, :N], s2[1, :N], da, db,
                       W_e1[:H], W_e1[H:], be1)

    w2b = jnp.broadcast_to(W_e2.reshape(H, 1), (H, 16))
    b2b = jnp.broadcast_to(b_e2.reshape(1), (16,))
    fx = _flux()(row2d, col2d, a_t, b_t, w2b, b2b)
    return fx.reshape(E_PAD)[:E]


# speedup vs baseline: 1.0294x; 1.0294x over previous
import functools

import jax
import jax.numpy as jnp
from jax import lax
from jax.experimental import pallas as pl
from jax.experimental.pallas import tpu as pltpu
from jax.experimental.pallas import tpu_sc as plsc

N = 50000
E = 1600000
F_IN = 128
H = 32
HH = H // 2

NC = 2
NS = 16
NW = NC * NS
CH = 128
E_ROWS = 12544
E_PAD = E_ROWS * CH
ROWS_PER_S = E_ROWS // NS
KB = 8
OUTER_S = ROWS_PER_S // KB
ROWS_PER_W = E_ROWS // NW
N_TILE = 3128
N_PAD = NS * N_TILE


@functools.lru_cache(maxsize=None)
def _mesh():
    return plsc.VectorSubcoreMesh(
        core_axis_name="c", subcore_axis_name="s", num_cores=NC, num_subcores=NS
    )


_RB = 1000


def _tc1_body(x_ref, wi_ref, bi_ref, wt_ref, wb_ref, b0_ref,
              p_ref, qlo_ref, qhi_ref):
    h0 = jnp.maximum(
        jnp.dot(x_ref[...], wi_ref[...], preferred_element_type=jnp.float32)
        + bi_ref[...], 0.0)
    p_ref[...] = jnp.dot(h0, wt_ref[...],
                         preferred_element_type=jnp.float32) + b0_ref[...]
    q = jnp.dot(h0, wb_ref[...], preferred_element_type=jnp.float32)
    qlo_ref[...] = q[:, :HH]
    qhi_ref[...] = q[:, HH:]


def _tc_mid_body(p_ref, sa_ref, sb_ref, da_ref, db_ref, wt_ref, wb_ref, b_ref,
                 p2_ref, qlo_ref, qhi_ref):
    deg = jnp.maximum(da_ref[...] + db_ref[...], 1.0)[:, 0:1]
    s = jnp.concatenate([sa_ref[...], sb_ref[...]], axis=1)
    h = jnp.maximum(p_ref[...] + s / deg, 0.0)
    p2_ref[...] = jnp.dot(h, wt_ref[...],
                          preferred_element_type=jnp.float32) + b_ref[...]
    q = jnp.dot(h, wb_ref[...], preferred_element_type=jnp.float32)
    qlo_ref[...] = q[:, :HH]
    qhi_ref[...] = q[:, HH:]


def _tc_fin_body(p_ref, sa_ref, sb_ref, da_ref, db_ref, wt_ref, wb_ref, b_ref,
                 a_ref, b2_ref):
    deg = jnp.maximum(da_ref[...] + db_ref[...], 1.0)[:, 0:1]
    s = jnp.concatenate([sa_ref[...], sb_ref[...]], axis=1)
    h = jnp.maximum(p_ref[...] + s / deg, 0.0)
    a_ref[...] = jnp.dot(h, wt_ref[...],
                         preferred_element_type=jnp.float32) + b_ref[...]
    b2_ref[...] = jnp.dot(h, wb_ref[...], preferred_element_type=jnp.float32)


def _node_spec(w):
    return pl.BlockSpec((_RB, w), lambda i: (i, 0))


def _full_spec(r, w):
    return pl.BlockSpec((r, w), lambda i: (0, 0))


def _tc1(x, wi, bi, wt, wb, b0):
    return pl.pallas_call(
        _tc1_body,
        grid=(N // _RB,),
        in_specs=[
            _node_spec(F_IN), _full_spec(F_IN, H), _full_spec(1, H),
            _full_spec(H, H), _full_spec(H, H), _full_spec(1, H),
        ],
        out_specs=[_node_spec(H), _node_spec(HH), _node_spec(HH)],
        out_shape=[
            jax.ShapeDtypeStruct((N, H), jnp.float32),
            jax.ShapeDtypeStruct((N, HH), jnp.float32),
            jax.ShapeDtypeStruct((N, HH), jnp.float32),
        ],
    )(x, wi, bi, wt, wb, b0)


def _tc_mid(p, sa, sb, da, db, wt, wb, b):
    return pl.pallas_call(
        _tc_mid_body,
        grid=(N // _RB,),
        in_specs=[
            _node_spec(H), _node_spec(HH), _node_spec(HH), _node_spec(8),
            _node_spec(8),
            _full_spec(H, H), _full_spec(H, H), _full_spec(1, H),
        ],
        out_specs=[_node_spec(H), _node_spec(HH), _node_spec(HH)],
        out_shape=[
            jax.ShapeDtypeStruct((N, H), jnp.float32),
            jax.ShapeDtypeStruct((N, HH), jnp.float32),
            jax.ShapeDtypeStruct((N, HH), jnp.float32),
        ],
    )(p, sa, sb, da, db, wt, wb, b)


def _tc_fin(p, sa, sb, da, db, wt, wb, b):
    return pl.pallas_call(
        _tc_fin_body,
        grid=(N // _RB,),
        in_specs=[
            _node_spec(H), _node_spec(HH), _node_spec(HH), _node_spec(8),
            _node_spec(8),
            _full_spec(H, H), _full_spec(H, H), _full_spec(1, H),
        ],
        out_specs=[_node_spec(H), _node_spec(H)],
        out_shape=[
            jax.ShapeDtypeStruct((N, H), jnp.float32),
            jax.ShapeDtypeStruct((N, H), jnp.float32),
        ],
    )(p, sa, sb, da, db, wt, wb, b)


DEPTH = 4
STEPS_S = 104
STEPS_F = 394


def _seg_body(row2d, col2d, q_lo, q_hi, z16, *refs):
    n = DEPTH
    s_out = refs[0]
    refs = refs[1:]
    ridx = refs[0:n]
    cidx = refs[n:2 * n]
    rows = refs[2 * n:3 * n]
    acc = refs[3 * n]
    refs = refs[3 * n + 1:]
    semi = refs[0:n]
    semg = refs[n:2 * n]
    sems = refs[2 * n:3 * n]

    c = lax.axis_index("c")
    s = lax.axis_index("s")
    lo = s * N_TILE
    pltpu.sync_copy(z16.at[pl.ds(lo, N_TILE)], acc.at[pl.ds(lo, N_TILE)])
    plsc.subcore_barrier()
    base = s * ROWS_PER_S

    def fire_idx(b, m):
        r0 = base + b * KB
        pltpu.async_copy(row2d.at[pl.ds(r0, KB)], ridx[m], semi[m])
        pltpu.async_copy(col2d.at[pl.ds(r0, KB)], cidx[m], semi[m])

    def wait_idx(m):
        pltpu.make_async_copy(row2d.at[pl.ds(base, KB)], ridx[m], semi[m]).wait()
        pltpu.make_async_copy(col2d.at[pl.ds(base, KB)], cidx[m], semi[m]).wait()

    def fire_gathers(m):
        @pl.when(c == 0)
        def _():
            for j in range(KB):
                pltpu.async_copy(q_lo.at[cidx[m].at[j]], rows[m].at[j], semg[m])

        @pl.when(c == 1)
        def _():
            for j in range(KB):
                pltpu.async_copy(q_hi.at[cidx[m].at[j]], rows[m].at[j], semg[m])

    def wait_gathers(m):
        for j in range(KB):
            pltpu.make_async_copy(
                q_lo.at[cidx[m].at[j]], rows[m].at[j], semg[m]).wait()

    def fire_scatters(m):
        for j in range(KB):
            pltpu.async_copy(rows[m].at[j], acc.at[ridx[m].at[j]], sems[m],
                             add=True)

    def wait_scatters(m):
        for j in range(KB):
            pltpu.make_async_copy(
                rows[m].at[j], acc.at[ridx[m].at[j]], sems[m]).wait()

    fire_idx(0, 0)

    def loop_body(k, carry):
        i0 = k * DEPTH
        for u in range(DEPTH):
            i = i0 + u
            m = u
            m1 = (u + 1) % DEPTH
            mp = (u - 1) % DEPTH

            @pl.when(jnp.logical_and(i >= 3, i <= OUTER_S + 2))
            def _():
                wait_scatters(m1)

            @pl.when(i < OUTER_S)
            def _():
                wait_idx(m)
                fire_gathers(m)

            @pl.when(i + 1 < OUTER_S)
            def _():
                fire_idx(i + 1, m1)

            @pl.when(jnp.logical_and(i >= 1, i <= OUTER_S))
            def _():
                wait_gathers(mp)
                fire_scatters(mp)
        return carry

    lax.fori_loop(0, STEPS_S // DEPTH, loop_body, 0)
    plsc.subcore_barrier()
    pltpu.sync_copy(acc.at[pl.ds(lo, N_TILE)], s_out.at[c, pl.ds(lo, N_TILE)])


def _seg_scratch():
    sc = []
    sc += [pltpu.VMEM((KB, CH), jnp.int32) for _ in range(DEPTH)]
    sc += [pltpu.VMEM((KB, CH), jnp.int32) for _ in range(DEPTH)]
    sc += [pltpu.VMEM((KB, CH, HH), jnp.float32) for _ in range(DEPTH)]
    sc += [pltpu.VMEM_SHARED((N_PAD, HH), jnp.float32)]
    sc += [pltpu.SemaphoreType.DMA for _ in range(3 * DEPTH)]
    return sc


@functools.lru_cache(maxsize=None)
def _segsum():
    return pl.kernel(
        _seg_body,
        out_type=[jax.ShapeDtypeStruct((NC, N_PAD, HH), jnp.float32)],
        mesh=_mesh(),
        compiler_params=pltpu.CompilerParams(use_tc_tiling_on_sc=False, needs_layout_passes=False),
        scratch_types=_seg_scratch(),
    )


OUTER_D = ROWS_PER_W // KB
STEPS_D = 52


def _deg_body(row2d, z4, ones4, d_out, *refs):
    n = DEPTH
    ridx = refs[0:n]
    ones_v = refs[n]
    dacc = refs[n + 1]
    semi = refs[n + 2:2 * n + 2]
    sems = refs[2 * n + 2:3 * n + 2]

    c = lax.axis_index("c")
    s = lax.axis_index("s")
    wid = s * NC + c
    lo = s * N_TILE
    pltpu.sync_copy(z4.at[pl.ds(lo, N_TILE)], dacc.at[pl.ds(lo, N_TILE)])
    pltpu.sync_copy(ones4, ones_v)
    plsc.subcore_barrier()
    base = wid * ROWS_PER_W

    def fire_idx(b, m):
        r0 = base + b * KB
        pltpu.async_copy(row2d.at[pl.ds(r0, KB)], ridx[m], semi[m])

    def wait_idx(m):
        pltpu.make_async_copy(row2d.at[pl.ds(base, KB)], ridx[m], semi[m]).wait()

    def fire_scatters(m):
        for j in range(KB):
            pltpu.async_copy(ones_v, dacc.at[ridx[m].at[j]], sems[m], add=True)

    def wait_scatters(m):
        for j in range(KB):
            pltpu.make_async_copy(ones_v, dacc.at[ridx[m].at[j]], sems[m]).wait()

    fire_idx(0, 0)

    def loop_body(k, carry):
        i0 = k * DEPTH
        for u in range(DEPTH):
            i = i0 + u
            m = u
            m1 = (u + 1) % DEPTH

            @pl.when(jnp.logical_and(i >= 3, i <= OUTER_D + 2))
            def _():
                wait_scatters(m1)

            @pl.when(i < OUTER_D)
            def _():
                wait_idx(m)
                fire_scatters(m)

            @pl.when(i + 1 < OUTER_D)
            def _():
                fire_idx(i + 1, m1)
        return carry

    lax.fori_loop(0, STEPS_D // DEPTH, loop_body, 0)
    plsc.subcore_barrier()
    pltpu.sync_copy(dacc.at[pl.ds(lo, N_TILE)], d_out.at[c, pl.ds(lo, N_TILE)])


@functools.lru_cache(maxsize=None)
def _deg():
    return pl.kernel(
        _deg_body,
        out_type=jax.ShapeDtypeStruct((NC, N_PAD, 8), jnp.float32),
        mesh=_mesh(),
        compiler_params=pltpu.CompilerParams(use_tc_tiling_on_sc=False, needs_layout_passes=False),
        scratch_types=(
            [pltpu.VMEM((KB, CH), jnp.int32) for _ in range(DEPTH)]
            + [pltpu.VMEM((CH, 8), jnp.float32)]
            + [pltpu.VMEM_SHARED((N_PAD, 8), jnp.float32)]
            + [pltpu.SemaphoreType.DMA for _ in range(2 * DEPTH)]
        ),
    )


KF = 2
NB_F = ROWS_PER_W // KF
STEPS_FB = 198
NG = CH // 16


def _flux_body(row2d, col2d, a_t, b_t, w2b, b2b, out,
               ridx0, ridx1, cidx0, cidx1,
               ra00, ra01, ra10, ra11, rb00, rb01, rb10, rb11,
               fb0, fb1, w2v, b2v,
               semi0, semi1, semg0, semg1, semw0, semw1):
    ridx = (ridx0, ridx1)
    cidx = (cidx0, cidx1)
    ra = ((ra00, ra01), (ra10, ra11))
    rb = ((rb00, rb01), (rb10, rb11))
    fbuf = (fb0, fb1)
    semi = (semi0, semi1)
    semg = (semg0, semg1)
    semw = (semw0, semw1)
    c = lax.axis_index("c")
    s = lax.axis_index("s")
    wid = s * NC + c
    pltpu.sync_copy(w2b, w2v)
    pltpu.sync_copy(b2b, b2v)
    base = wid * ROWS_PER_W
    iotas = [lax.iota(jnp.int32, 16) + g * 16 for g in range(NG)]
    b2 = b2v[...]

    def fire_idx(kb, p):
        r0 = base + kb * KF
        pltpu.async_copy(row2d.at[pl.ds(r0, KF)], ridx[p], semi[p])
        pltpu.async_copy(col2d.at[pl.ds(r0, KF)], cidx[p], semi[p])

    def wait_idx(p):
        pltpu.make_async_copy(row2d.at[pl.ds(base, KF)], ridx[p], semi[p]).wait()
        pltpu.make_async_copy(col2d.at[pl.ds(base, KF)], cidx[p], semi[p]).wait()

    def fire_gathers(p):
        for r in range(KF):
            pltpu.async_copy(a_t.at[ridx[p].at[r]], ra[p][r], semg[p])
            pltpu.async_copy(b_t.at[cidx[p].at[r]], rb[p][r], semg[p])

    def wait_gathers(p):
        for r in range(KF):
            pltpu.make_async_copy(a_t.at[ridx[p].at[r]], ra[p][r],
                                  semg[p]).wait()
            pltpu.make_async_copy(b_t.at[cidx[p].at[r]], rb[p][r],
                                  semg[p]).wait()

    def compute_batch(p, kb):
        for r in range(KF):
            rar = ra[p][r]
            rbr = rb[p][r]

            def jbody(t, accs):
                out_accs = list(accs)
                for d in range(2):
                    jj = t * 2 + d
                    w2j = w2v[jj]
                    jv = jnp.full((16,), jj, jnp.int32)
                    for g in range(NG):
                        va = plsc.load_gather(rar, [iotas[g], jv])
                        vb = plsc.load_gather(rbr, [iotas[g], jv])
                        sv = jnp.maximum(va + vb, 0.0)
                        out_accs[g] = out_accs[g] + sv * w2j
                return tuple(out_accs)

            accs = lax.fori_loop(0, H // 2, jbody, (b2,) * NG)
            for g in range(NG):
                fbuf[p][r, pl.ds(g * 16, 16)] = accs[g]
        pltpu.async_copy(fbuf[p], out.at[pl.ds(base + kb * KF, KF)], semw[p])

    def wait_write(p):
        pltpu.make_async_copy(fbuf[p], out.at[pl.ds(base, KF)], semw[p]).wait()

    fire_idx(0, 0)

    def loop_body(k0, carry):
        for u in range(2):
            k = k0 * 2 + u
            p = u
            pp = 1 - u

            @pl.when(k < NB_F)
            def _():
                wait_idx(p)
                fire_gathers(p)

            @pl.when(jnp.logical_and(k >= 1, k <= NB_F))
            def _():
                wait_gathers(pp)

                @pl.when(k >= 3)
                def _():
                    wait_write(pp)
                compute_batch(pp, k - 1)

            @pl.when(k + 1 < NB_F)
            def _():
                fire_idx(k + 1, pp)
        return carry

    lax.fori_loop(0, STEPS_FB // 2, loop_body, 0)
    wait_write(0)
    wait_write(1)


@functools.lru_cache(maxsize=None)
def _flux():
    return pl.kernel(
        _flux_body,
        out_type=jax.ShapeDtypeStruct((E_ROWS, CH), jnp.float32),
        mesh=_mesh(),
        compiler_params=pltpu.CompilerParams(use_tc_tiling_on_sc=False, needs_layout_passes=False),
        scratch_types=(
            [pltpu.VMEM((KF, CH), jnp.int32) for _ in range(4)]
            + [pltpu.VMEM((CH, H), jnp.float32) for _ in range(8)]
            + [pltpu.VMEM((KF, CH), jnp.float32) for _ in range(2)]
            + [pltpu.VMEM((H, 16), jnp.float32), pltpu.VMEM((16,), jnp.float32)]
            + [pltpu.SemaphoreType.DMA for _ in range(6)]
        ),
    )


def kernel(node_features, edge_index, W_in, b_in, W_up0, b_up0,
           W_up1, b_up1, W_e1, b_e1, W_e2, b_e2):
    row = edge_index[0].astype(jnp.int32)
    col = edge_index[1].astype(jnp.int32)
    pad = E_PAD - E
    row2d = jnp.concatenate(
        [row, jnp.full((pad,), N, jnp.int32)]).reshape(E_ROWS, CH)
    col2d = jnp.concatenate(
        [col, jnp.zeros((pad,), jnp.int32)]).reshape(E_ROWS, CH)

    z16 = jnp.zeros((N_PAD, HH), jnp.float32)
    z4 = jnp.zeros((N_PAD, 8), jnp.float32)
    ones4 = jnp.ones((CH, 8), jnp.float32)

    bi = b_in.reshape(1, H)
    b0 = b_up0.reshape(1, H)
    b1 = b_up1.reshape(1, H)
    be1 = b_e1.reshape(1, H)

    p1, q1lo, q1hi = _tc1(node_features, W_in, bi, W_up0[:H], W_up0[H:], b0)
    dpart = _deg()(row2d, z4, ones4)
    (s1,) = _segsum()(row2d, col2d, q1lo, q1hi, z16)
    da, db = dpart[0, :N], dpart[1, :N]
    p2, q2lo, q2hi = _tc_mid(p1, s1[0, :N], s1[1, :N], da, db,
                             W_up1[:H], W_up1[H:], b1)
    (s2,) = _segsum()(row2d, col2d, q2lo, q2hi, z16)
    a_t, b_t = _tc_fin(p2, s2[0, :N], s2[1, :N], da, db,
                       W_e1[:H], W_e1[H:], be1)

    w2b = jnp.broadcast_to(W_e2.reshape(H, 1), (H, 16))
    b2b = jnp.broadcast_to(b_e2.reshape(1), (16,))
    fx = _flux()(row2d, col2d, a_t, b_t, w2b, b2b)
    return fx.reshape(E_PAD)[:E]


# speedup vs baseline: 1.0956x; 1.0643x over previous
import functools

import jax
import jax.numpy as jnp
from jax import lax
from jax.experimental import pallas as pl
from jax.experimental.pallas import tpu as pltpu
from jax.experimental.pallas import tpu_sc as plsc

N = 50000
E = 1600000
F_IN = 128
H = 32
HH = H // 2

NC = 2
NS = 16
NW = NC * NS
CH = 128
E_ROWS = 12544
E_PAD = E_ROWS * CH
ROWS_PER_S = E_ROWS // NS
KB = 8
OUTER_S = ROWS_PER_S // KB
ROWS_PER_W = E_ROWS // NW
N_TILE = 3128
N_PAD = NS * N_TILE


@functools.lru_cache(maxsize=None)
def _mesh():
    return plsc.VectorSubcoreMesh(
        core_axis_name="c", subcore_axis_name="s", num_cores=NC, num_subcores=NS
    )


_RB = 1000


def _tc1_body(x_ref, wi_ref, bi_ref, wt_ref, wb_ref, b0_ref,
              p_ref, qlo_ref, qhi_ref):
    h0 = jnp.maximum(
        jnp.dot(x_ref[...], wi_ref[...], preferred_element_type=jnp.float32)
        + bi_ref[...], 0.0)
    p_ref[...] = jnp.dot(h0, wt_ref[...],
                         preferred_element_type=jnp.float32) + b0_ref[...]
    q = jnp.dot(h0, wb_ref[...], preferred_element_type=jnp.float32)
    qlo_ref[...] = q[:, :HH]
    qhi_ref[...] = q[:, HH:]


def _tc_mid_body(p_ref, sa_ref, sb_ref, da_ref, db_ref, wt_ref, wb_ref, b_ref,
                 p2_ref, qlo_ref, qhi_ref):
    deg = jnp.maximum(da_ref[...] + db_ref[...], 1.0)[:, 0:1]
    s = jnp.concatenate([sa_ref[...], sb_ref[...]], axis=1)
    h = jnp.maximum(p_ref[...] + s / deg, 0.0)
    p2_ref[...] = jnp.dot(h, wt_ref[...],
                          preferred_element_type=jnp.float32) + b_ref[...]
    q = jnp.dot(h, wb_ref[...], preferred_element_type=jnp.float32)
    qlo_ref[...] = q[:, :HH]
    qhi_ref[...] = q[:, HH:]


def _tc_fin_body(p_ref, sa_ref, sb_ref, da_ref, db_ref, wt_ref, wb_ref, b_ref,
                 a_ref, b2_ref):
    deg = jnp.maximum(da_ref[...] + db_ref[...], 1.0)[:, 0:1]
    s = jnp.concatenate([sa_ref[...], sb_ref[...]], axis=1)
    h = jnp.maximum(p_ref[...] + s / deg, 0.0)
    a_ref[...] = jnp.dot(h, wt_ref[...],
                         preferred_element_type=jnp.float32) + b_ref[...]
    b2_ref[...] = jnp.dot(h, wb_ref[...], preferred_element_type=jnp.float32)


def _node_spec(w):
    return pl.BlockSpec((_RB, w), lambda i: (i, 0))


def _full_spec(r, w):
    return pl.BlockSpec((r, w), lambda i: (0, 0))


def _tc1(x, wi, bi, wt, wb, b0):
    return pl.pallas_call(
        _tc1_body,
        grid=(N // _RB,),
        in_specs=[
            _node_spec(F_IN), _full_spec(F_IN, H), _full_spec(1, H),
            _full_spec(H, H), _full_spec(H, H), _full_spec(1, H),
        ],
        out_specs=[_node_spec(H), _node_spec(HH), _node_spec(HH)],
        out_shape=[
            jax.ShapeDtypeStruct((N, H), jnp.float32),
            jax.ShapeDtypeStruct((N, HH), jnp.float32),
            jax.ShapeDtypeStruct((N, HH), jnp.float32),
        ],
    )(x, wi, bi, wt, wb, b0)


def _tc_mid(p, sa, sb, da, db, wt, wb, b):
    return pl.pallas_call(
        _tc_mid_body,
        grid=(N // _RB,),
        in_specs=[
            _node_spec(H), _node_spec(HH), _node_spec(HH), _node_spec(8),
            _node_spec(8),
            _full_spec(H, H), _full_spec(H, H), _full_spec(1, H),
        ],
        out_specs=[_node_spec(H), _node_spec(HH), _node_spec(HH)],
        out_shape=[
            jax.ShapeDtypeStruct((N, H), jnp.float32),
            jax.ShapeDtypeStruct((N, HH), jnp.float32),
            jax.ShapeDtypeStruct((N, HH), jnp.float32),
        ],
    )(p, sa, sb, da, db, wt, wb, b)


def _tc_fin(p, sa, sb, da, db, wt, wb, b):
    return pl.pallas_call(
        _tc_fin_body,
        grid=(N // _RB,),
        in_specs=[
            _node_spec(H), _node_spec(HH), _node_spec(HH), _node_spec(8),
            _node_spec(8),
            _full_spec(H, H), _full_spec(H, H), _full_spec(1, H),
        ],
        out_specs=[_node_spec(H), _node_spec(H)],
        out_shape=[
            jax.ShapeDtypeStruct((N, H), jnp.float32),
            jax.ShapeDtypeStruct((N, H), jnp.float32),
        ],
    )(p, sa, sb, da, db, wt, wb, b)


DEPTH = 4
STEPS_S = 104
STEPS_F = 394


def _seg_body(row2d, col2d, q_lo, q_hi, z16, *refs):
    n = DEPTH
    s_out = refs[0]
    refs = refs[1:]
    ridx = refs[0:n]
    cidx = refs[n:2 * n]
    rows = refs[2 * n:3 * n]
    acc = refs[3 * n]
    refs = refs[3 * n + 1:]
    semi = refs[0:n]
    semg = refs[n:2 * n]
    sems = refs[2 * n:3 * n]

    c = lax.axis_index("c")
    s = lax.axis_index("s")
    lo = s * N_TILE
    pltpu.sync_copy(z16.at[pl.ds(lo, N_TILE)], acc.at[pl.ds(lo, N_TILE)])
    plsc.subcore_barrier()
    base = s * ROWS_PER_S

    def fire_idx(b, m):
        r0 = base + b * KB
        pltpu.async_copy(row2d.at[pl.ds(r0, KB)], ridx[m], semi[m])
        pltpu.async_copy(col2d.at[pl.ds(r0, KB)], cidx[m], semi[m])

    def wait_idx(m):
        pltpu.make_async_copy(row2d.at[pl.ds(base, KB)], ridx[m], semi[m]).wait()
        pltpu.make_async_copy(col2d.at[pl.ds(base, KB)], cidx[m], semi[m]).wait()

    def fire_gathers(m):
        @pl.when(c == 0)
        def _():
            for j in range(KB):
                pltpu.async_copy(q_lo.at[cidx[m].at[j]], rows[m].at[j], semg[m])

        @pl.when(c == 1)
        def _():
            for j in range(KB):
                pltpu.async_copy(q_hi.at[cidx[m].at[j]], rows[m].at[j], semg[m])

    def wait_gathers(m):
        for j in range(KB):
            pltpu.make_async_copy(
                q_lo.at[cidx[m].at[j]], rows[m].at[j], semg[m]).wait()

    def fire_scatters(m):
        for j in range(KB):
            pltpu.async_copy(rows[m].at[j], acc.at[ridx[m].at[j]], sems[m],
                             add=True)

    def wait_scatters(m):
        for j in range(KB):
            pltpu.make_async_copy(
                rows[m].at[j], acc.at[ridx[m].at[j]], sems[m]).wait()

    fire_idx(0, 0)

    def loop_body(k, carry):
        i0 = k * DEPTH
        for u in range(DEPTH):
            i = i0 + u
            m = u
            m1 = (u + 1) % DEPTH
            mp = (u - 1) % DEPTH

            @pl.when(jnp.logical_and(i >= 3, i <= OUTER_S + 2))
            def _():
                wait_scatters(m1)

            @pl.when(i < OUTER_S)
            def _():
                wait_idx(m)
                fire_gathers(m)

            @pl.when(i + 1 < OUTER_S)
            def _():
                fire_idx(i + 1, m1)

            @pl.when(jnp.logical_and(i >= 1, i <= OUTER_S))
            def _():
                wait_gathers(mp)
                fire_scatters(mp)
        return carry

    lax.fori_loop(0, STEPS_S // DEPTH, loop_body, 0)
    plsc.subcore_barrier()
    pltpu.sync_copy(acc.at[pl.ds(lo, N_TILE)], s_out.at[c, pl.ds(lo, N_TILE)])


def _seg_scratch():
    sc = []
    sc += [pltpu.VMEM((KB, CH), jnp.int32) for _ in range(DEPTH)]
    sc += [pltpu.VMEM((KB, CH), jnp.int32) for _ in range(DEPTH)]
    sc += [pltpu.VMEM((KB, CH, HH), jnp.float32) for _ in range(DEPTH)]
    sc += [pltpu.VMEM_SHARED((N_PAD, HH), jnp.float32)]
    sc += [pltpu.SemaphoreType.DMA for _ in range(3 * DEPTH)]
    return sc


@functools.lru_cache(maxsize=None)
def _segsum():
    return pl.kernel(
        _seg_body,
        out_type=[jax.ShapeDtypeStruct((NC, N_PAD, HH), jnp.float32)],
        mesh=_mesh(),
        compiler_params=pltpu.CompilerParams(use_tc_tiling_on_sc=False, needs_layout_passes=False),
        scratch_types=_seg_scratch(),
    )


OUTER_D = ROWS_PER_W // KB
STEPS_D = 52


def _deg_body(row2d, z4, ones4, d_out, *refs):
    n = DEPTH
    ridx = refs[0:n]
    ones_v = refs[n]
    dacc = refs[n + 1]
    semi = refs[n + 2:2 * n + 2]
    sems = refs[2 * n + 2:3 * n + 2]

    c = lax.axis_index("c")
    s = lax.axis_index("s")
    wid = s * NC + c
    lo = s * N_TILE
    pltpu.sync_copy(z4.at[pl.ds(lo, N_TILE)], dacc.at[pl.ds(lo, N_TILE)])
    pltpu.sync_copy(ones4, ones_v)
    plsc.subcore_barrier()
    base = wid * ROWS_PER_W

    def fire_idx(b, m):
        r0 = base + b * KB
        pltpu.async_copy(row2d.at[pl.ds(r0, KB)], ridx[m], semi[m])

    def wait_idx(m):
        pltpu.make_async_copy(row2d.at[pl.ds(base, KB)], ridx[m], semi[m]).wait()

    def fire_scatters(m):
        for j in range(KB):
            pltpu.async_copy(ones_v, dacc.at[ridx[m].at[j]], sems[m], add=True)

    def wait_scatters(m):
        for j in range(KB):
            pltpu.make_async_copy(ones_v, dacc.at[ridx[m].at[j]], sems[m]).wait()

    fire_idx(0, 0)

    def loop_body(k, carry):
        i0 = k * DEPTH
        for u in range(DEPTH):
            i = i0 + u
            m = u
            m1 = (u + 1) % DEPTH

            @pl.when(jnp.logical_and(i >= 3, i <= OUTER_D + 2))
            def _():
                wait_scatters(m1)

            @pl.when(i < OUTER_D)
            def _():
                wait_idx(m)
                fire_scatters(m)

            @pl.when(i + 1 < OUTER_D)
            def _():
                fire_idx(i + 1, m1)
        return carry

    lax.fori_loop(0, STEPS_D // DEPTH, loop_body, 0)
    plsc.subcore_barrier()
    pltpu.sync_copy(dacc.at[pl.ds(lo, N_TILE)], d_out.at[c, pl.ds(lo, N_TILE)])


@functools.lru_cache(maxsize=None)
def _deg():
    return pl.kernel(
        _deg_body,
        out_type=jax.ShapeDtypeStruct((NC, N_PAD, 8), jnp.float32),
        mesh=_mesh(),
        compiler_params=pltpu.CompilerParams(use_tc_tiling_on_sc=False, needs_layout_passes=False),
        scratch_types=(
            [pltpu.VMEM((KB, CH), jnp.int32) for _ in range(DEPTH)]
            + [pltpu.VMEM((CH, 8), jnp.float32)]
            + [pltpu.VMEM_SHARED((N_PAD, 8), jnp.float32)]
            + [pltpu.SemaphoreType.DMA for _ in range(2 * DEPTH)]
        ),
    )


KF = 2
NB_F = ROWS_PER_W // KF
STEPS_FB = 198
NG = CH // 16


def _flux_body(row2d, col2d, a_t, b_t, w2b, b2b, out,
               ridx0, ridx1, cidx0, cidx1,
               ra00, ra01, ra10, ra11, rb00, rb01, rb10, rb11,
               fb0, fb1, w2v, b2v,
               semi0, semi1, semg0, semg1, semw0, semw1):
    ridx = (ridx0, ridx1)
    cidx = (cidx0, cidx1)
    ra = ((ra00, ra01), (ra10, ra11))
    rb = ((rb00, rb01), (rb10, rb11))
    fbuf = (fb0, fb1)
    semi = (semi0, semi1)
    semg = (semg0, semg1)
    semw = (semw0, semw1)
    c = lax.axis_index("c")
    s = lax.axis_index("s")
    wid = s * NC + c
    pltpu.sync_copy(w2b, w2v)
    pltpu.sync_copy(b2b, b2v)
    base = wid * ROWS_PER_W
    iotas = [lax.iota(jnp.int32, 16) + g * 16 for g in range(NG)]
    b2 = b2v[...]

    def fire_idx(kb, p):
        r0 = base + kb * KF
        pltpu.async_copy(row2d.at[pl.ds(r0, KF)], ridx[p], semi[p])
        pltpu.async_copy(col2d.at[pl.ds(r0, KF)], cidx[p], semi[p])

    def wait_idx(p):
        pltpu.make_async_copy(row2d.at[pl.ds(base, KF)], ridx[p], semi[p]).wait()
        pltpu.make_async_copy(col2d.at[pl.ds(base, KF)], cidx[p], semi[p]).wait()

    def fire_gathers(p):
        for r in range(KF):
            pltpu.async_copy(a_t.at[ridx[p].at[r]], ra[p][r], semg[p])
            pltpu.async_copy(b_t.at[cidx[p].at[r]], rb[p][r], semg[p])

    def wait_gathers(p):
        for r in range(KF):
            pltpu.make_async_copy(a_t.at[ridx[p].at[r]], ra[p][r],
                                  semg[p]).wait()
            pltpu.make_async_copy(b_t.at[cidx[p].at[r]], rb[p][r],
                                  semg[p]).wait()

    def compute_batch(p, kb):
        for r in range(KF):
            rar = ra[p][r]
            rbr = rb[p][r]

            def jbody(t, accs):
                out_accs = list(accs)
                for d in range(2):
                    jj = t * 2 + d
                    jv = jnp.full((16,), jj, jnp.int32)
                    w2j = plsc.load_gather(w2v, [jv, iotas[0]])
                    for g in range(NG):
                        va = plsc.load_gather(rar, [iotas[g], jv])
                        vb = plsc.load_gather(rbr, [iotas[g], jv])
                        sv = jnp.maximum(va + vb, 0.0)
                        out_accs[g] = out_accs[g] + sv * w2j
                return tuple(out_accs)

            accs = lax.fori_loop(0, H // 2, jbody, (b2,) * NG)
            for g in range(NG):
                fbuf[p][r, pl.ds(g * 16, 16)] = accs[g]
        pltpu.async_copy(fbuf[p], out.at[pl.ds(base + kb * KF, KF)], semw[p])

    def wait_write(p):
        pltpu.make_async_copy(fbuf[p], out.at[pl.ds(base, KF)], semw[p]).wait()

    fire_idx(0, 0)

    def loop_body(k0, carry):
        for u in range(2):
            k = k0 * 2 + u
            p = u
            pp = 1 - u

            @pl.when(k < NB_F)
            def _():
                wait_idx(p)
                fire_gathers(p)

            @pl.when(jnp.logical_and(k >= 1, k <= NB_F))
            def _():
                wait_gathers(pp)

                @pl.when(k >= 3)
                def _():
                    wait_write(pp)
                compute_batch(pp, k - 1)

            @pl.when(k + 1 < NB_F)
            def _():
                fire_idx(k + 1, pp)
        return carry

    lax.fori_loop(0, STEPS_FB // 2, loop_body, 0)
    wait_write(0)
    wait_write(1)


@functools.lru_cache(maxsize=None)
def _flux():
    return pl.kernel(
        _flux_body,
        out_type=jax.ShapeDtypeStruct((E_ROWS, CH), jnp.float32),
        mesh=_mesh(),
        compiler_params=pltpu.CompilerParams(use_tc_tiling_on_sc=False, needs_layout_passes=False),
        scratch_types=(
            [pltpu.VMEM((KF, CH), jnp.int32) for _ in range(4)]
            + [pltpu.VMEM((CH, H), jnp.float32) for _ in range(8)]
            + [pltpu.VMEM((KF, CH), jnp.float32) for _ in range(2)]
            + [pltpu.VMEM((H, 16), jnp.float32), pltpu.VMEM((16,), jnp.float32)]
            + [pltpu.SemaphoreType.DMA for _ in range(6)]
        ),
    )


def kernel(node_features, edge_index, W_in, b_in, W_up0, b_up0,
           W_up1, b_up1, W_e1, b_e1, W_e2, b_e2):
    row = edge_index[0].astype(jnp.int32)
    col = edge_index[1].astype(jnp.int32)
    pad = E_PAD - E
    row2d = jnp.concatenate(
        [row, jnp.full((pad,), N, jnp.int32)]).reshape(E_ROWS, CH)
    col2d = jnp.concatenate(
        [col, jnp.zeros((pad,), jnp.int32)]).reshape(E_ROWS, CH)

    z16 = jnp.zeros((N_PAD, HH), jnp.float32)
    z4 = jnp.zeros((N_PAD, 8), jnp.float32)
    ones4 = jnp.ones((CH, 8), jnp.float32)

    bi = b_in.reshape(1, H)
    b0 = b_up0.reshape(1, H)
    b1 = b_up1.reshape(1, H)
    be1 = b_e1.reshape(1, H)

    p1, q1lo, q1hi = _tc1(node_features, W_in, bi, W_up0[:H], W_up0[H:], b0)
    dpart = _deg()(row2d, z4, ones4)
    (s1,) = _segsum()(row2d, col2d, q1lo, q1hi, z16)
    da, db = dpart[0, :N], dpart[1, :N]
    p2, q2lo, q2hi = _tc_mid(p1, s1[0, :N], s1[1, :N], da, db,
                             W_up1[:H], W_up1[H:], b1)
    (s2,) = _segsum()(row2d, col2d, q2lo, q2hi, z16)
    a_t, b_t = _tc_fin(p2, s2[0, :N], s2[1, :N], da, db,
                       W_e1[:H], W_e1[H:], be1)

    w2b = jnp.broadcast_to(W_e2.reshape(H, 1), (H, 16))
    b2b = jnp.broadcast_to(b_e2.reshape(1), (16,))
    fx = _flux()(row2d, col2d, a_t, b_t, w2b, b2b)
    return fx.reshape(E_PAD)[:E]
